# baseline (speedup 1.0000x reference)
import functools
import jax
import jax.numpy as jnp
from jax import lax
from jax.experimental import pallas as pl
from jax.experimental.pallas import tpu as pltpu
from jax.experimental.pallas import tpu_sc as plsc

_ANCHORS = (1.08, 1.19, 3.42, 4.41, 6.63, 11.38, 9.42, 5.11, 16.62, 10.52)
_NA = 5
_NC = 8
_NH = 32
_NW = 32
_NT = 50
_OBJ = 10.0
_THRESH = 0.6
_CHUNK = 10
_NCH = 7 + _NC
_ROWS = _NT * _NCH
_RPAD = 768

_EMULATE_SC = False


def _anchor_match(gw, gl, aw, al):
    inter_a = (jnp.minimum(aw, gw[..., None, :])
               * jnp.minimum(al, gl[..., None, :]))
    union_a = (aw * al) + (gw * gl)[..., None, :] - inter_a
    iou_a = inter_a / union_a
    best = jnp.max(iou_a, axis=-2)
    a_iota = lax.broadcasted_iota(jnp.int32, iou_a.shape, iou_a.ndim - 2)
    bn = jnp.min(jnp.where(iou_a == best[..., None, :], a_iota, _NA + 1),
                 axis=-2)
    return bn, best


def _idx_body(t_ref, a_ref, s_ref, off_ref, out_ref):
    aw = a_ref[0][None, :, None]
    al = a_ref[1][None, :, None]
    gx = t_ref[:, 1, :] * _NW
    gy = t_ref[:, 2, :] * _NH
    gw = t_ref[:, 3, :] * _NW
    gl = t_ref[:, 4, :] * _NH
    bn, _ = _anchor_match(gw, gl, aw, al)
    gi = gx.astype(jnp.int32)
    gj = gy.astype(jnp.int32)
    cellp = gj * _NW + gi
    base = (bn * (_NCH * _NH * _NW) + cellp).astype(jnp.float32)
    idx_f = lax.dot_general(base, s_ref[...], (((1,), (0,)), ((), ())),
                            precision=lax.Precision.HIGHEST)
    out_ref[...] = (idx_f + off_ref[0][None, :] + 0.5).astype(jnp.int32)


def _gather_body(o_hbm, idx_hbm, out_hbm, blk_v, idx_v, out_v, sem):
    wid = lax.axis_index("s") * 2 + lax.axis_index("c")
    h = pltpu.async_copy(o_hbm.at[wid], blk_v, sem)
    pltpu.sync_copy(idx_hbm.at[wid], idx_v)
    h.wait()
    for k in range(_RPAD // 16):
        iv = idx_v[pl.ds(k * 16, 16)]
        out_v[pl.ds(k * 16, 16)] = plsc.load_gather(blk_v, [iv])
    pltpu.sync_copy(out_v, out_hbm.at[wid, 0])


def _dense_body(o_ref, t_ref, a_ref, out_ref):
    f32 = jnp.float32
    i32 = jnp.int32
    o = o_ref[0]
    tgt = t_ref[0]
    aw = a_ref[0]
    al = a_ref[1]

    o3 = o.reshape(_NA, _NCH, _NH * _NW)
    xs = jax.nn.sigmoid(o3[:, 0, :])
    ys = jax.nn.sigmoid(o3[:, 1, :])
    confs = jax.nn.sigmoid(o3[:, 6, :])
    p = lax.broadcasted_iota(i32, (_NA, _NH * _NW), 1)
    grid_x = (p & (_NW - 1)).astype(f32)
    grid_y = (p >> 5).astype(f32)
    px = xs + grid_x
    py = ys + grid_y
    pw = jnp.exp(o3[:, 2, :]) * aw[:, None]
    plh = jnp.exp(o3[:, 3, :]) * al[:, None]
    xl = px - pw * 0.5
    xr = px + pw * 0.5
    yl = py - plh * 0.5
    yr = py + plh * 0.5
    parea = pw * plh

    gx = tgt[1] * _NW
    gy = tgt[2] * _NH
    gw = tgt[3] * _NW
    gl = tgt[4] * _NH
    gxl = gx - gw * 0.5
    gxr = gx + gw * 0.5
    gyl = gy - gl * 0.5
    gyr = gy + gl * 0.5
    garea = gw * gl

    tt = lax.broadcasted_iota(i32, (_NT, _NT), 0)
    ss = lax.broadcasted_iota(i32, (_NT, _NT), 1)
    zero_seen = jnp.any((ss <= tt) & (tgt[1] == 0.0)[None, :], axis=1)
    valid = jnp.logical_not(zero_seen)

    c_t = _THRESH / (1.0 + _THRESH)
    cga = jnp.where(valid, c_t * garea, 1.0e30)
    m = jnp.full((_NA, _NH * _NW), -1.0e30, f32)
    for c0 in range(0, _NT, _CHUNK):
        sl = slice(c0, c0 + _CHUNK)
        cw = (jnp.minimum(xr[None], gxr[sl, None, None])
              - jnp.maximum(xl[None], gxl[sl, None, None]))
        ch = (jnp.minimum(yr[None], gyr[sl, None, None])
              - jnp.maximum(yl[None], gyl[sl, None, None]))
        inter = jnp.maximum(cw, 0.0) * jnp.maximum(ch, 0.0)
        m = jnp.maximum(m, jnp.max(inter - cga[sl, None, None], axis=0))
    noobj = (m <= c_t * parea).astype(f32)
    out_ref[:, :, :] = (0.5 * jnp.sum(confs * confs * noobj))[None, None, None]


def _corr_body(g_ref, t_ref, a_ref, out_ref):
    f32 = jnp.float32
    i32 = jnp.int32
    g = g_ref[0, 0]
    tgt = t_ref[0]
    aw = a_ref[0]
    al = a_ref[1]

    gx = tgt[1] * _NW
    gy = tgt[2] * _NH
    gw = tgt[3] * _NW
    gl = tgt[4] * _NH
    gxl = gx - gw * 0.5
    gxr = gx + gw * 0.5
    gyl = gy - gl * 0.5
    gyr = gy + gl * 0.5
    garea = gw * gl

    tt = lax.broadcasted_iota(i32, (_NT, _NT), 0)
    ss = lax.broadcasted_iota(i32, (_NT, _NT), 1)
    zero_seen = jnp.any((ss <= tt) & (tgt[1] == 0.0)[None, :], axis=1)
    valid = jnp.logical_not(zero_seen)

    bn, best = _anchor_match(gw, gl, aw[:, None], al[:, None])
    do = valid & (best > 0.0)
    gi = gx.astype(i32)
    gj = gy.astype(i32)
    cellp = gj * _NW + gi
    slot = bn * (_NH * _NW) + cellp
    later = ss > tt
    same = slot[None, :] == slot[:, None]
    clobbered = jnp.any(later & same & do[None, :], axis=1)
    win = do & jnp.logical_not(clobbered)

    def vch(c):
        return g[c * _NT:(c + 1) * _NT]
    xg = jax.nn.sigmoid(vch(0))
    yg = jax.nn.sigmoid(vch(1))
    wg = vch(2)
    lg = vch(3)
    img = vch(4)
    reg = vch(5)
    cg = jax.nn.sigmoid(vch(6))
    cls_g = jnp.concatenate([vch(7 + c)[None] for c in range(_NC)], axis=0)

    a_iota = lax.broadcasted_iota(i32, (_NA, _NT), 0)
    a_onehot = (a_iota == bn[None, :]).astype(f32)
    awb = jnp.sum(a_onehot * aw[:, None], axis=0)
    alb = jnp.sum(a_onehot * al[:, None], axis=0)

    gif = gi.astype(f32)
    gjf = gj.astype(f32)
    pxg = xg + gif
    pyg = yg + gjf
    pwg = jnp.exp(wg) * awb
    plg = jnp.exp(lg) * alb
    parea_t = pwg * plg
    pxl = pxg - pwg * 0.5
    pxr = pxg + pwg * 0.5
    pyl = pyg - plg * 0.5
    pyr = pyg + plg * 0.5

    c_t = _THRESH / (1.0 + _THRESH)
    cga = jnp.where(valid, c_t * garea, 1.0e30)
    cwm = (jnp.minimum(pxr[:, None], gxr[None, :])
           - jnp.maximum(pxl[:, None], gxl[None, :]))
    chm = (jnp.minimum(pyr[:, None], gyr[None, :])
           - jnp.maximum(pyl[:, None], gyl[None, :]))
    interm = jnp.maximum(cwm, 0.0) * jnp.maximum(chm, 0.0)
    mloc = jnp.max(interm - cga[None, :], axis=1)
    noobjg = (mloc <= c_t * parea_t).astype(f32)

    tx = gx - gif
    ty = gy - gjf
    gw_s = jnp.where(do, gw, 1.0)
    gl_s = jnp.where(do, gl, 1.0)
    tw = jnp.log(gw_s / awb)
    tl = jnp.log(gl_s / alb)
    tim = tgt[5]
    tre = tgt[6]

    coord = ((xg - tx) ** 2 + (yg - ty) ** 2 + (wg - tw) ** 2
             + (lg - tl) ** 2 + (img - tim) ** 2 + (reg - tre) ** 2)

    cw2 = jnp.minimum(gxr, pxr) - jnp.maximum(gxl, pxl)
    ch2 = jnp.minimum(gyr, pyr) - jnp.maximum(gyl, pyl)
    ca2 = cw2 * ch2
    confv = jnp.where((cw2 <= 0.0) | (ch2 <= 0.0), 0.0,
                      ca2 / (garea + parea_t - ca2))

    cmax = jnp.max(cls_g, axis=0)
    lse = cmax + jnp.log(jnp.sum(jnp.exp(cls_g - cmax[None]), axis=0))
    c_iota = lax.broadcasted_iota(i32, (_NC, _NT), 0)
    tcls = tgt[0].astype(i32)
    picked = jnp.sum(jnp.where(c_iota == tcls[None, :], cls_g, 0.0), axis=0)

    per_t = (0.5 * coord
             + 0.5 * _OBJ * _OBJ * (cg - confv) ** 2
             - 0.5 * noobjg * cg * cg
             + (lse - picked))
    out_ref[:, :, :] = jnp.sum(jnp.where(win, per_t, 0.0))[None, None, None]


def kernel(output, target):
    nB = output.shape[0]
    f32 = jnp.float32
    anc = jnp.asarray(_ANCHORS, f32).reshape(_NA, 2).T

    lanes = jnp.arange(_RPAD, dtype=jnp.int32)
    s_mat = jnp.where(
        lanes[None, :] < _ROWS,
        (lanes[None, :] % _NT
         == jnp.arange(_NT, dtype=jnp.int32)[:, None]).astype(f32),
        0.0)
    off = jnp.where(lanes < _ROWS, (lanes // _NT) * (_NH * _NW), 0).astype(f32)
    idx = pl.pallas_call(
        _idx_body,
        in_specs=[
            pl.BlockSpec((nB, 7, _NT), lambda: (0, 0, 0)),
            pl.BlockSpec((2, _NA), lambda: (0, 0)),
            pl.BlockSpec((_NT, _RPAD), lambda: (0, 0)),
            pl.BlockSpec((1, _RPAD), lambda: (0, 0)),
        ],
        out_specs=pl.BlockSpec((nB, _RPAD), lambda: (0, 0)),
        out_shape=jax.ShapeDtypeStruct((nB, _RPAD), jnp.int32),
    )(target.transpose(0, 2, 1), anc, s_mat, off[None, :])

    o3d = output.reshape(nB, _NA * _NCH, _NH * _NW)
    if _EMULATE_SC:
        gathered = jnp.take_along_axis(
            o3d.reshape(nB, _NA * _NCH * _NH * _NW), idx, axis=1
        ).reshape(nB, 1, _RPAD)
    else:
        mesh = plsc.VectorSubcoreMesh(core_axis_name="c",
                                      subcore_axis_name="s")
        gather_k = functools.partial(
            pl.kernel, mesh=mesh,
            compiler_params=pltpu.CompilerParams(needs_layout_passes=False),
            out_type=jax.ShapeDtypeStruct((nB, 1, _RPAD), f32),
            scratch_types=[
                pltpu.VMEM((_NA * _NCH * _NH * _NW,), f32),
                pltpu.VMEM((_RPAD,), jnp.int32),
                pltpu.VMEM((_RPAD,), f32),
                pltpu.SemaphoreType.DMA,
            ],
        )(_gather_body)
        gathered = gather_k(
            output.reshape(nB, _NA * _NCH * _NH * _NW), idx)

    t3 = target.transpose(0, 2, 1)
    dense = pl.pallas_call(
        _dense_body,
        grid=(nB,),
        in_specs=[
            pl.BlockSpec((1, _NA * _NCH, _NH * _NW), lambda b: (b, 0, 0)),
            pl.BlockSpec((1, 7, _NT), lambda b: (b, 0, 0)),
            pl.BlockSpec((2, _NA), lambda b: (0, 0)),
        ],
        out_specs=pl.BlockSpec((1, 1, 1), lambda b: (b, 0, 0)),
        out_shape=jax.ShapeDtypeStruct((nB, 1, 1), f32),
    )(output.reshape(nB, _NA * _NCH, _NH * _NW), t3, anc)

    corr = pl.pallas_call(
        _corr_body,
        grid=(nB,),
        in_specs=[
            pl.BlockSpec((1, 1, _RPAD), lambda b: (b, 0, 0)),
            pl.BlockSpec((1, 7, _NT), lambda b: (b, 0, 0)),
            pl.BlockSpec((2, _NA), lambda b: (0, 0)),
        ],
        out_specs=pl.BlockSpec((1, 1, 1), lambda b: (b, 0, 0)),
        out_shape=jax.ShapeDtypeStruct((nB, 1, 1), f32),
    )(gathered, t3, anc)

    return jnp.sum(dense) + jnp.sum(corr)


# speedup vs baseline: 2.5852x; 2.5852x over previous
import jax
import jax.numpy as jnp
from jax import lax
from jax.experimental import pallas as pl
from jax.experimental.pallas import tpu as pltpu

_ANCHORS = (1.08, 1.19, 3.42, 4.41, 6.63, 11.38, 9.42, 5.11, 16.62, 10.52)
_NA = 5
_NC = 8
_NH = 32
_NW = 32
_NT = 50
_OBJ = 10.0
_THRESH = 0.6
_CHUNK = 10


def _batch_loss(o, tgt, aw, al):
    f32 = jnp.float32
    i32 = jnp.int32

    o3 = o.reshape(_NA, 7 + _NC, _NH * _NW)
    xs = jax.nn.sigmoid(o3[:, 0, :])
    ys = jax.nn.sigmoid(o3[:, 1, :])
    ws = o3[:, 2, :]
    ls = o3[:, 3, :]
    ims = o3[:, 4, :]
    res = o3[:, 5, :]
    confs = jax.nn.sigmoid(o3[:, 6, :])

    p = lax.broadcasted_iota(i32, (_NA, _NH * _NW), 1)
    grid_x = (p & (_NW - 1)).astype(f32)
    grid_y = (p >> 5).astype(f32)
    px = xs + grid_x
    py = ys + grid_y
    pw = jnp.exp(ws) * aw[:, None]
    plh = jnp.exp(ls) * al[:, None]
    xl = px - pw * 0.5
    xr = px + pw * 0.5
    yl = py - plh * 0.5
    yr = py + plh * 0.5
    parea = pw * plh

    gx = tgt[1] * _NW
    gy = tgt[2] * _NH
    gw = tgt[3] * _NW
    gl = tgt[4] * _NH
    gxl = gx - gw * 0.5
    gxr = gx + gw * 0.5
    gyl = gy - gl * 0.5
    gyr = gy + gl * 0.5
    garea = gw * gl

    tt = lax.broadcasted_iota(i32, (_NT, _NT), 0)
    ss = lax.broadcasted_iota(i32, (_NT, _NT), 1)
    zero_seen = jnp.any((ss <= tt) & (tgt[1] == 0.0)[None, :], axis=1)
    valid = jnp.logical_not(zero_seen)

    c_t = _THRESH / (1.0 + _THRESH)
    cga = jnp.where(valid, c_t * garea, 1.0e30)
    m = jnp.full((_NA, _NH * _NW), -1.0e30, f32)
    for c0 in range(0, _NT, _CHUNK):
        sl = slice(c0, c0 + _CHUNK)
        cw = (jnp.minimum(xr[None], gxr[sl, None, None])
              - jnp.maximum(xl[None], gxl[sl, None, None]))
        ch = (jnp.minimum(yr[None], gyr[sl, None, None])
              - jnp.maximum(yl[None], gyl[sl, None, None]))
        inter = jnp.maximum(cw, 0.0) * jnp.maximum(ch, 0.0)
        m = jnp.maximum(m, jnp.max(inter - cga[sl, None, None], axis=0))
    noobj = (m <= c_t * parea).astype(f32)
    dense_conf = 0.5 * jnp.sum(confs * confs * noobj)

    inter_a = (jnp.minimum(aw[:, None], gw[None, :])
               * jnp.minimum(al[:, None], gl[None, :]))
    union_a = (aw * al)[:, None] + garea[None, :] - inter_a
    iou_a = inter_a / union_a
    best = jnp.max(iou_a, axis=0)
    a_iota = lax.broadcasted_iota(i32, (_NA, _NT), 0)
    bn = jnp.min(jnp.where(iou_a == best[None, :], a_iota, _NA + 1), axis=0)
    do = valid & (best > 0.0)

    gi = gx.astype(i32)
    gj = gy.astype(i32)
    cellp = gj * _NW + gi
    slot = bn * (_NH * _NW) + cellp

    later = ss > tt
    same = slot[None, :] == slot[:, None]
    clobbered = jnp.any(later & same & do[None, :], axis=1)
    win = do & jnp.logical_not(clobbered)

    vmat = jnp.concatenate(
        [xs, ys, ws, ls, ims, res, confs, noobj,
         o3[:, 7, :], o3[:, 8, :], o3[:, 9, :], o3[:, 10, :],
         o3[:, 11, :], o3[:, 12, :], o3[:, 13, :], o3[:, 14, :]],
        axis=0)
    p_iota = lax.broadcasted_iota(i32, (_NT, _NH * _NW), 1)
    onehot_p = (p_iota == cellp[:, None]).astype(f32)
    g80 = lax.dot_general(vmat, onehot_p, (((1,), (1,)), ((), ())),
                          precision=lax.Precision.HIGHEST)
    a_onehot = (a_iota == bn[None, :]).astype(f32)
    g = jnp.sum(g80.reshape(16, _NA, _NT) * a_onehot[None], axis=1)

    xg, yg, wg, lg, img, reg, cg, noobjg = (g[0], g[1], g[2], g[3],
                                            g[4], g[5], g[6], g[7])
    cls_g = g[8:16]

    awb = jnp.sum(a_onehot * aw[:, None], axis=0)
    alb = jnp.sum(a_onehot * al[:, None], axis=0)

    gif = gi.astype(f32)
    gjf = gj.astype(f32)
    tx = gx - gif
    ty = gy - gjf
    gw_s = jnp.where(do, gw, 1.0)
    gl_s = jnp.where(do, gl, 1.0)
    tw = jnp.log(gw_s / awb)
    tl = jnp.log(gl_s / alb)
    tim = tgt[5]
    tre = tgt[6]

    coord = ((xg - tx) ** 2 + (yg - ty) ** 2 + (wg - tw) ** 2
             + (lg - tl) ** 2 + (img - tim) ** 2 + (reg - tre) ** 2)

    pxg = xg + gif
    pyg = yg + gjf
    pwg = jnp.exp(wg) * awb
    plg = jnp.exp(lg) * alb
    cw2 = jnp.minimum(gxr, pxg + pwg * 0.5) - jnp.maximum(gxl, pxg - pwg * 0.5)
    ch2 = jnp.minimum(gyr, pyg + plg * 0.5) - jnp.maximum(gyl, pyg - plg * 0.5)
    ca2 = cw2 * ch2
    confv = jnp.where((cw2 <= 0.0) | (ch2 <= 0.0), 0.0,
                      ca2 / (garea + pwg * plg - ca2))

    cmax = jnp.max(cls_g, axis=0)
    lse = cmax + jnp.log(jnp.sum(jnp.exp(cls_g - cmax[None]), axis=0))
    c_iota = lax.broadcasted_iota(i32, (_NC, _NT), 0)
    tcls = tgt[0].astype(i32)
    picked = jnp.sum(jnp.where(c_iota == tcls[None, :], cls_g, 0.0), axis=0)

    per_t = (0.5 * coord
             + 0.5 * _OBJ * _OBJ * (cg - confv) ** 2
             - 0.5 * noobjg * cg * cg
             + (lse - picked))
    sparse_loss = jnp.sum(jnp.where(win, per_t, 0.0))

    return dense_conf + sparse_loss


_BPS = 1


def _loss_body(o_ref, t_ref, a_ref, out_ref):
    aw = a_ref[0]
    al = a_ref[1]
    total = 0.0
    for i in range(_BPS):
        total = total + _batch_loss(o_ref[i], t_ref[i], aw, al)
    out_ref[:, :, :] = total[None, None, None]


def kernel(output, target):
    nB = output.shape[0]
    o = output.reshape(nB, _NA * (7 + _NC), _NH * _NW)
    t = target.transpose(0, 2, 1)
    anc = jnp.asarray(_ANCHORS, jnp.float32).reshape(_NA, 2).T
    steps = nB // _BPS
    res = pl.pallas_call(
        _loss_body,
        grid=(steps,),
        in_specs=[
            pl.BlockSpec((_BPS, _NA * (7 + _NC), _NH * _NW),
                         lambda b: (b, 0, 0)),
            pl.BlockSpec((_BPS, 7, _NT), lambda b: (b, 0, 0)),
            pl.BlockSpec((2, _NA), lambda b: (0, 0)),
        ],
        out_specs=pl.BlockSpec((1, 1, 1), lambda b: (b, 0, 0)),
        out_shape=jax.ShapeDtypeStruct((steps, 1, 1), jnp.float32),
    )(o, t, anc)
    return jnp.sum(res)


# speedup vs baseline: 2.7977x; 1.0822x over previous
import jax
import jax.numpy as jnp
from jax import lax
from jax.experimental import pallas as pl
from jax.experimental.pallas import tpu as pltpu

_ANCHORS = (1.08, 1.19, 3.42, 4.41, 6.63, 11.38, 9.42, 5.11, 16.62, 10.52)
_NA = 5
_NC = 8
_NH = 32
_NW = 32
_NT = 50
_OBJ = 10.0
_THRESH = 0.6
_CHUNK = 10


def _batch_loss(o, tgt, aw, al):
    f32 = jnp.float32
    i32 = jnp.int32

    o3 = o.reshape(_NA, 7 + _NC, _NH * _NW)
    xs = jax.nn.sigmoid(o3[:, 0, :])
    ys = jax.nn.sigmoid(o3[:, 1, :])
    ws = o3[:, 2, :]
    ls = o3[:, 3, :]
    ims = o3[:, 4, :]
    res = o3[:, 5, :]
    confs = jax.nn.sigmoid(o3[:, 6, :])

    p = lax.broadcasted_iota(i32, (_NA, _NH * _NW), 1)
    grid_x = (p & (_NW - 1)).astype(f32)
    grid_y = (p >> 5).astype(f32)
    px = xs + grid_x
    py = ys + grid_y
    pw = jnp.exp(ws) * aw[:, None]
    plh = jnp.exp(ls) * al[:, None]
    xl = px - pw * 0.5
    xr = px + pw * 0.5
    yl = py - plh * 0.5
    yr = py + plh * 0.5
    parea = pw * plh

    gx = tgt[1] * _NW
    gy = tgt[2] * _NH
    gw = tgt[3] * _NW
    gl = tgt[4] * _NH
    gxl = gx - gw * 0.5
    gxr = gx + gw * 0.5
    gyl = gy - gl * 0.5
    gyr = gy + gl * 0.5
    garea = gw * gl

    tt = lax.broadcasted_iota(i32, (_NT, _NT), 0)
    ss = lax.broadcasted_iota(i32, (_NT, _NT), 1)
    zero_seen = jnp.any((ss <= tt) & (tgt[1] == 0.0)[None, :], axis=1)
    valid = jnp.logical_not(zero_seen)

    c_t = _THRESH / (1.0 + _THRESH)
    cga = jnp.where(valid, c_t * garea, 1.0e30)
    m = jnp.full((_NA, _NH * _NW), -1.0e30, f32)
    for c0 in range(0, _NT, _CHUNK):
        sl = slice(c0, c0 + _CHUNK)
        cw = (jnp.minimum(xr[None], gxr[sl, None, None])
              - jnp.maximum(xl[None], gxl[sl, None, None]))
        ch = (jnp.minimum(yr[None], gyr[sl, None, None])
              - jnp.maximum(yl[None], gyl[sl, None, None]))
        inter = jnp.maximum(cw, 0.0) * jnp.maximum(ch, 0.0)
        m = jnp.maximum(m, jnp.max(inter - cga[sl, None, None], axis=0))
    noobj = (m <= c_t * parea).astype(f32)
    dense_conf = 0.5 * jnp.sum(confs * confs * noobj)

    inter_a = (jnp.minimum(aw[:, None], gw[None, :])
               * jnp.minimum(al[:, None], gl[None, :]))
    union_a = (aw * al)[:, None] + garea[None, :] - inter_a
    iou_a = inter_a / union_a
    best = jnp.max(iou_a, axis=0)
    a_iota = lax.broadcasted_iota(i32, (_NA, _NT), 0)
    bn = jnp.min(jnp.where(iou_a == best[None, :], a_iota, _NA + 1), axis=0)
    do = valid & (best > 0.0)

    gi = gx.astype(i32)
    gj = gy.astype(i32)
    cellp = gj * _NW + gi
    slot = bn * (_NH * _NW) + cellp

    later = ss > tt
    same = slot[None, :] == slot[:, None]
    clobbered = jnp.any(later & same & do[None, :], axis=1)
    win = do & jnp.logical_not(clobbered)

    vmat = jnp.concatenate(
        [xs, ys, ws, ls, ims, res, confs, noobj], axis=0)
    cmat = jnp.concatenate(
        [o3[:, 7, :], o3[:, 8, :], o3[:, 9, :], o3[:, 10, :],
         o3[:, 11, :], o3[:, 12, :], o3[:, 13, :], o3[:, 14, :]],
        axis=0)
    p_iota = lax.broadcasted_iota(i32, (_NT, _NH * _NW), 1)
    onehot_p = (p_iota == cellp[:, None]).astype(f32)
    g40 = lax.dot_general(vmat, onehot_p, (((1,), (1,)), ((), ())),
                          precision=lax.Precision.HIGHEST)
    c40 = lax.dot_general(cmat, onehot_p, (((1,), (1,)), ((), ())),
                          precision=lax.Precision.DEFAULT)
    a_onehot = (a_iota == bn[None, :]).astype(f32)
    g = jnp.sum(g40.reshape(8, _NA, _NT) * a_onehot[None], axis=1)
    gc = jnp.sum(c40.reshape(8, _NA, _NT) * a_onehot[None], axis=1)

    xg, yg, wg, lg, img, reg, cg, noobjg = (g[0], g[1], g[2], g[3],
                                            g[4], g[5], g[6], g[7])
    cls_g = gc

    awb = jnp.sum(a_onehot * aw[:, None], axis=0)
    alb = jnp.sum(a_onehot * al[:, None], axis=0)

    gif = gi.astype(f32)
    gjf = gj.astype(f32)
    tx = gx - gif
    ty = gy - gjf
    gw_s = jnp.where(do, gw, 1.0)
    gl_s = jnp.where(do, gl, 1.0)
    tw = jnp.log(gw_s / awb)
    tl = jnp.log(gl_s / alb)
    tim = tgt[5]
    tre = tgt[6]

    coord = ((xg - tx) ** 2 + (yg - ty) ** 2 + (wg - tw) ** 2
             + (lg - tl) ** 2 + (img - tim) ** 2 + (reg - tre) ** 2)

    pxg = xg + gif
    pyg = yg + gjf
    pwg = jnp.exp(wg) * awb
    plg = jnp.exp(lg) * alb
    cw2 = jnp.minimum(gxr, pxg + pwg * 0.5) - jnp.maximum(gxl, pxg - pwg * 0.5)
    ch2 = jnp.minimum(gyr, pyg + plg * 0.5) - jnp.maximum(gyl, pyg - plg * 0.5)
    ca2 = cw2 * ch2
    confv = jnp.where((cw2 <= 0.0) | (ch2 <= 0.0), 0.0,
                      ca2 / (garea + pwg * plg - ca2))

    cmax = jnp.max(cls_g, axis=0)
    lse = cmax + jnp.log(jnp.sum(jnp.exp(cls_g - cmax[None]), axis=0))
    c_iota = lax.broadcasted_iota(i32, (_NC, _NT), 0)
    tcls = tgt[0].astype(i32)
    picked = jnp.sum(jnp.where(c_iota == tcls[None, :], cls_g, 0.0), axis=0)

    per_t = (0.5 * coord
             + 0.5 * _OBJ * _OBJ * (cg - confv) ** 2
             - 0.5 * noobjg * cg * cg
             + (lse - picked))
    sparse_loss = jnp.sum(jnp.where(win, per_t, 0.0))

    return dense_conf + sparse_loss


_BPS = 1


def _loss_body(o_ref, t_ref, a_ref, out_ref):
    aw = a_ref[0]
    al = a_ref[1]
    total = 0.0
    for i in range(_BPS):
        total = total + _batch_loss(o_ref[i], t_ref[i], aw, al)
    out_ref[:, :, :] = total[None, None, None]


def kernel(output, target):
    nB = output.shape[0]
    o = output.reshape(nB, _NA * (7 + _NC), _NH * _NW)
    t = target.transpose(0, 2, 1)
    anc = jnp.asarray(_ANCHORS, jnp.float32).reshape(_NA, 2).T
    steps = nB // _BPS
    res = pl.pallas_call(
        _loss_body,
        grid=(steps,),
        in_specs=[
            pl.BlockSpec((_BPS, _NA * (7 + _NC), _NH * _NW),
                         lambda b: (b, 0, 0)),
            pl.BlockSpec((_BPS, 7, _NT), lambda b: (b, 0, 0)),
            pl.BlockSpec((2, _NA), lambda b: (0, 0)),
        ],
        out_specs=pl.BlockSpec((1, 1, 1), lambda b: (b, 0, 0)),
        out_shape=jax.ShapeDtypeStruct((steps, 1, 1), jnp.float32),
    )(o, t, anc)
    return jnp.sum(res)
